# baseline (speedup 1.0000x reference)
import functools
import math

import jax
import jax.numpy as jnp
from jax import lax
from jax.experimental import pallas as pl
from jax.experimental.pallas import tpu as pltpu
from jax.experimental.pallas import tpu_sc as plsc

VOCAB = 1000000
EMB = 64
SCALE = math.sqrt(EMB)

NC = 2
NS = 16
NW = NC * NS

B = 4096 * 200
BPW = B // NW
CH = 128
NCHUNK = BPW // CH


def _emb_kernel(table_hbm, idx_hbm, out_hbm, idx_v,
                gb0, gb1, ob0, ob1, gsem0, gsem1, osem0, osem1):
    wid = lax.axis_index("s") * NC + lax.axis_index("c")
    base = wid * BPW
    gbufs = (gb0, gb1)
    obufs = (ob0, ob1)
    gsems = (gsem0, gsem1)
    osems = (osem0, osem1)

    pltpu.sync_copy(idx_hbm.at[pl.ds(wid * NCHUNK, NCHUNK)], idx_v)

    for b in range(2):
        pltpu.async_copy(table_hbm.at[idx_v.at[b]], gbufs[b], gsems[b])

    def group_body(g, carry):
        for b in range(2):
            j = g * 2 + b
            gb, ob, gsem, osem = gbufs[b], obufs[b], gsems[b], osems[b]
            pltpu.make_async_copy(table_hbm.at[idx_v.at[j]], gb, gsem).wait()

            @pl.when(j >= 2)
            def _():
                pltpu.make_async_copy(
                    ob, out_hbm.at[pl.ds(base + (j - 2) * CH, CH)], osem
                ).wait()

            def scale_body(i, c):
                for q in range(EMB // 16):
                    s = pl.ds(q * 16, 16)
                    ob[i, s] = gb[i, s] * SCALE
                return c

            lax.fori_loop(0, CH, scale_body, 0, unroll=8)
            pltpu.async_copy(ob, out_hbm.at[pl.ds(base + j * CH, CH)], osem)

            @pl.when(j + 2 < NCHUNK)
            def _():
                pltpu.async_copy(table_hbm.at[idx_v.at[j + 2]], gb, gsem)

        return carry

    lax.fori_loop(0, NCHUNK // 2, group_body, 0)

    for b in range(2):
        j = NCHUNK - 2 + b
        pltpu.make_async_copy(
            obufs[b], out_hbm.at[pl.ds(base + j * CH, CH)], osems[b]
        ).wait()


@jax.jit
def _emb_lookup(idx2d, table):
    mesh = plsc.VectorSubcoreMesh(core_axis_name="c", subcore_axis_name="s")
    fn = functools.partial(
        pl.kernel,
        out_type=jax.ShapeDtypeStruct((B, EMB), jnp.float32),
        mesh=mesh,
        scratch_types=[
            pltpu.VMEM((NCHUNK, CH), jnp.int32),
            pltpu.VMEM((CH, EMB), jnp.float32),
            pltpu.VMEM((CH, EMB), jnp.float32),
            pltpu.VMEM((CH, EMB), jnp.float32),
            pltpu.VMEM((CH, EMB), jnp.float32),
            pltpu.SemaphoreType.DMA,
            pltpu.SemaphoreType.DMA,
            pltpu.SemaphoreType.DMA,
            pltpu.SemaphoreType.DMA,
        ],
        compiler_params=pltpu.CompilerParams(use_tc_tiling_on_sc=False),
    )(_emb_kernel)
    return fn(table, idx2d)


def kernel(tokens, table):
    idx2d = tokens.reshape(-1).astype(jnp.int32).reshape(NW * NCHUNK, CH)
    out = _emb_lookup(idx2d, table)
    return out.reshape(tokens.shape[0], tokens.shape[1], EMB)
